# baseline (speedup 1.0000x reference)
import functools

import jax
import jax.numpy as jnp
from jax import lax
from jax.experimental import pallas as pl
from jax.experimental.pallas import tpu as pltpu
from jax.experimental.pallas import tpu_sc as plsc

N = 10000
E = 320000
H = 128
NB = 32
MAXZ = 100

NC = 2
NS = 16
NW = NC * NS
EPW = E // NW
K = 200
NCH = EPW // K
KS = 200
NCHS = EPW // KS
CK = 1000
NSTR = 640
NP = NS * NSTR

f32 = jnp.float32


def _preproc_body(post_ref, z_ref, batn_ref, batt_ref, emb_ref,
                  we_ref, wp_ref, lib_ref, wc_ref, wr_ref, b1_ref,
                  x0_ref, prt_ref, a0_ref, b0_ref):
    post = post_ref[...]
    z = z_ref[...]
    batn = batn_ref[...]
    batt = batt_ref[...]

    oh_nb = (lax.broadcasted_iota(jnp.int32, (N, NB), 1) == batn).astype(f32)
    oh_bn = (lax.broadcasted_iota(jnp.int32, (NB, N), 0) == batt).astype(f32)

    cnt = jnp.sum(oh_nb, axis=0, keepdims=True)
    centT = jnp.dot(post, oh_nb, preferred_element_type=f32)
    centT = centT / jnp.maximum(cnt, 1.0)
    cbT = jnp.dot(centT, oh_bn, preferred_element_type=f32)
    prT = post - cbT
    prt_ref[...] = prT

    zoh = (lax.broadcasted_iota(jnp.int32, (N, MAXZ), 1) == z).astype(f32)
    ew = jnp.dot(emb_ref[...], we_ref[...], preferred_element_type=f32)
    ez = jnp.dot(zoh, ew, preferred_element_type=f32)
    xp = lax.dot_general(prT, wp_ref[...], (((0,), (0,)), ((), ())),
                         preferred_element_type=f32)
    x0 = ez + xp + lib_ref[...]
    x0_ref[...] = x0
    a0_ref[...] = jnp.dot(x0, wc_ref[...], preferred_element_type=f32) + b1_ref[...]
    b0_ref[...] = jnp.dot(x0, wr_ref[...], preferred_element_type=f32)


def _preproc(posT, z2, bat_n, bat_t, emb, we, wp, lib, wc, wr, b1):
    return pl.pallas_call(
        _preproc_body,
        out_shape=(
            jax.ShapeDtypeStruct((N, H), f32),
            jax.ShapeDtypeStruct((3, N), f32),
            jax.ShapeDtypeStruct((N, H), f32),
            jax.ShapeDtypeStruct((N, H), f32),
        ),
    )(posT, z2, bat_n, bat_t, emb, we, wp, lib, wc, wr, b1)


def _d2_body(prx_hbm, pry_hbm, prz_hbm, col_hbm, row_hbm, zn_hbm,
             d2_hbm, cnt_hbm,
             prx, pry, prz, colv, rowv, d2v, cntv):
    cid = lax.axis_index("c")
    sid = lax.axis_index("s")
    wid = sid * NC + cid
    base = wid * EPW

    pltpu.sync_copy(prx_hbm, prx)
    pltpu.sync_copy(pry_hbm, pry)
    pltpu.sync_copy(prz_hbm, prz)
    pltpu.sync_copy(col_hbm.at[pl.ds(base, EPW)], colv)
    pltpu.sync_copy(row_hbm.at[pl.ds(base, EPW)], rowv)
    pltpu.sync_copy(zn_hbm, cntv)

    ones16 = jnp.full((16,), 1.0, f32)

    def step(g, _):
        s = pl.ds(g * 16, 16)
        ic = colv[s]
        ir = rowv[s]
        dx = plsc.load_gather(prx, [ic]) - plsc.load_gather(prx, [ir])
        dy = plsc.load_gather(pry, [ic]) - plsc.load_gather(pry, [ir])
        dz = plsc.load_gather(prz, [ic]) - plsc.load_gather(prz, [ir])
        d2v[s] = dx * dx + dy * dy + dz * dz
        plsc.addupdate_scatter(cntv, [ir], ones16)
        return 0

    lax.fori_loop(0, EPW // 16, step, 0)
    pltpu.sync_copy(d2v, d2_hbm.at[pl.ds(base, EPW)])
    pltpu.sync_copy(cntv, cnt_hbm.at[pl.ds(wid * N, N)])


def _d2_counts(prx, pry, prz, col, row):
    zn = jnp.zeros((N,), f32)
    mesh = plsc.VectorSubcoreMesh(core_axis_name="c", subcore_axis_name="s")
    return pl.kernel(
        _d2_body,
        out_type=(
            jax.ShapeDtypeStruct((E,), f32),
            jax.ShapeDtypeStruct((NW * N,), f32),
        ),
        mesh=mesh,
        scratch_types=[
            pltpu.VMEM((N,), f32),
            pltpu.VMEM((N,), f32),
            pltpu.VMEM((N,), f32),
            pltpu.VMEM((EPW,), jnp.int32),
            pltpu.VMEM((EPW,), jnp.int32),
            pltpu.VMEM((EPW,), f32),
            pltpu.VMEM((N,), f32),
        ],
        compiler_params=pltpu.CompilerParams(needs_layout_passes=False),
    )(prx, pry, prz, col, row, zn)


def _sqrt_body(d2_ref, cnt_ref, o_ref, inv_ref):
    o_ref[...] = jnp.sqrt(d2_ref[...] + 1e-12)
    c = lax.dot_general(cnt_ref[...], jnp.ones((NW, 1), f32),
                        (((0,), (0,)), ((), ())),
                        preferred_element_type=f32)
    inv_ref[...] = 1.0 / jnp.maximum(c, 1.0)


def _dist_inv(d2, cntNW):
    d2m = d2.reshape(E // 128, 128)
    out, inv = pl.pallas_call(
        _sqrt_body,
        out_shape=(
            jax.ShapeDtypeStruct((E // 128, 128), f32),
            jax.ShapeDtypeStruct((N, 1), f32),
        ),
    )(d2m, cntNW)
    return out.reshape(E), inv


HP = H // 2
bf16 = jnp.bfloat16


KG = 80
NCHG = EPW // KG


def _gather_body(a_hbm, b_hbm, col_hbm, row_hbm, dist_hbm, wd_hbm,
                 p_hbm,
                 colw, roww, distw, av0, bv0, av1, bv1,
                 wdv, sg0, sg1, so0, so1):
    cid = lax.axis_index("c")
    sid = lax.axis_index("s")
    wid = sid * NC + cid

    av = (av0, av1)
    bv = (bv0, bv1)
    sg = (sg0, sg1)
    so = (so0, so1)

    pltpu.sync_copy(wd_hbm, wdv)
    pltpu.sync_copy(col_hbm.at[pl.ds(wid * EPW, EPW)], colw)
    pltpu.sync_copy(row_hbm.at[pl.ds(wid * EPW, EPW)], roww)
    pltpu.sync_copy(dist_hbm.at[pl.ds(wid * EPW, EPW)], distw)

    def fetch(ch, b):
        base = ch * KG
        pltpu.async_copy(a_hbm.at[colw.at[pl.ds(base, KG)]], av[b], sg[b])
        pltpu.async_copy(b_hbm.at[roww.at[pl.ds(base, KG)]], bv[b], sg[b])

    def edge(b, e, ro, co, d):
        for gg in range(HP // 16):
            s_lo = pl.ds(gg * 32, 16)
            s_hi = pl.ds(gg * 32 + 16, 16)
            r_lo = jnp.maximum(
                av[b][e, s_lo] + bv[b][e, s_lo] + d * wdv[s_lo], 0.0)
            r_hi = jnp.maximum(
                av[b][e, s_hi] + bv[b][e, s_hi] + d * wdv[s_hi], 0.0)
            bv[b][ro, pl.ds(co + gg * 16, 16)] = plsc.bitcast(
                plsc.pack(r_lo, r_hi, format=plsc.PackFormat.INTERLEAVED),
                f32)

    def compute(b, ch):
        def group(g, _):
            dvec = distw[pl.ds(ch * KG + g * 16, 16)]
            for j in range(16):
                edge(b, g * 16 + j, g * 8 + j // 2, (j % 2) * 64, dvec[j])
            return 0

        lax.fori_loop(0, KG // 16, group, 0)

    def wait_gather(b):
        pltpu.make_async_copy(
            a_hbm.at[colw.at[pl.ds(0, KG)]], av[b], sg[b]).wait()
        pltpu.make_async_copy(
            b_hbm.at[roww.at[pl.ds(0, KG)]], bv[b], sg[b]).wait()

    def wait_out(b):
        pltpu.make_async_copy(bv[b].at[pl.ds(0, KG // 2)],
                              p_hbm.at[pl.ds(0, KG // 2)], so[b]).wait()

    def emit(b, ch):
        base2 = pl.multiple_of((wid * EPW + ch * KG) // 2, 8)
        pltpu.async_copy(bv[b].at[pl.ds(0, KG // 2)],
                         p_hbm.at[pl.ds(base2, KG // 2)], so[b])

    fetch(0, 0)

    def step2(g2, _):
        for b in range(2):
            ch = g2 * 2 + b
            nb = 1 - b

            @pl.when(ch + 1 < NCHG)
            def _():
                @pl.when(ch >= 1)
                def _():
                    wait_out(nb)
                fetch(ch + 1, nb)

            wait_gather(b)
            compute(b, ch)
            emit(b, ch)
        return 0

    lax.fori_loop(0, NCHG // 2, step2, 0)
    if NCHG % 2:
        wait_gather(0)
        compute(0, NCHG - 1)
        emit(0, NCHG - 1)
    wait_out(0)
    wait_out(1)


def _gather_layer(A, B, col, row, dist, wd):
    mesh = plsc.VectorSubcoreMesh(core_axis_name="c", subcore_axis_name="s")
    return pl.kernel(
        _gather_body,
        out_type=jax.ShapeDtypeStruct((E // 2, H), f32),
        mesh=mesh,
        scratch_types=[
            pltpu.VMEM((EPW,), jnp.int32),
            pltpu.VMEM((EPW,), jnp.int32),
            pltpu.VMEM((EPW,), f32),
            pltpu.VMEM((KG, H), f32),
            pltpu.VMEM((KG, H), f32),
            pltpu.VMEM((KG, H), f32),
            pltpu.VMEM((KG, H), f32),
            pltpu.VMEM((H,), f32),
            pltpu.SemaphoreType.DMA,
            pltpu.SemaphoreType.DMA,
            pltpu.SemaphoreType.DMA,
            pltpu.SemaphoreType.DMA,
        ],
        compiler_params=pltpu.CompilerParams(needs_layout_passes=False),
    )(A, B, col, row, dist, wd)


BE = 3200


def _mm_body(p_ref, w_ref, b_ref, o_ref):
    o_ref[...] = jnp.maximum(
        jnp.dot(p_ref[...].astype(f32), w_ref[...],
                preferred_element_type=f32) + b_ref[...],
        0.0)


def _msg_mlp2(P, w, b):
    return pl.pallas_call(
        _mm_body,
        grid=(E // BE,),
        in_specs=[
            pl.BlockSpec((BE, H), lambda i: (i, 0)),
            pl.BlockSpec((H, H), lambda i: (0, 0)),
            pl.BlockSpec((1, H), lambda i: (0, 0)),
        ],
        out_specs=pl.BlockSpec((BE, H), lambda i: (i, 0)),
        out_shape=jax.ShapeDtypeStruct((E, H), f32),
    )(P, w, b.reshape(1, H))


def _scatter_body(m_hbm, row_hbm, zrows_hbm,
                  s_hbm,
                  rowv, mv, shared):
    cid = lax.axis_index("c")
    sid = lax.axis_index("s")
    wid = sid * NC + cid

    pltpu.sync_copy(zrows_hbm, shared.at[pl.ds(sid * NSTR, NSTR)])
    plsc.subcore_barrier()

    def chunk(ch, _):
        base = wid * EPW + ch * KS
        pltpu.sync_copy(row_hbm.at[pl.ds(base, KS)], rowv)
        pltpu.sync_copy(m_hbm.at[pl.ds(base, KS)], mv)
        pltpu.sync_copy(mv, shared.at[rowv], add=True)
        return 0

    lax.fori_loop(0, NCHS, chunk, 0)
    plsc.subcore_barrier()
    pltpu.sync_copy(shared.at[pl.ds(sid * NSTR, NSTR)],
                    s_hbm.at[cid, pl.ds(sid * NSTR, NSTR)])


def _scatter_layer(M, row):
    zrows = jnp.zeros((NSTR, H), f32)
    mesh = plsc.VectorSubcoreMesh(core_axis_name="c", subcore_axis_name="s")
    return pl.kernel(
        _scatter_body,
        out_type=jax.ShapeDtypeStruct((NC, NP, H), f32),
        mesh=mesh,
        scratch_types=[
            pltpu.VMEM((KS,), jnp.int32),
            pltpu.VMEM((KS, H), f32),
            pltpu.VMEM_SHARED((NP, H), f32),
        ],
    )(M, row, zrows)


NBK = 2000


def _update_body(x_ref, s0_ref, s1_ref, inv_ref,
                 uwx_ref, uwm_ref, ub_ref, wc_ref, wr_ref, b1_ref,
                 xo_ref, ao_ref, bo_ref):
    sm = (s0_ref[0] + s1_ref[0]) * inv_ref[...]
    x = x_ref[...]
    xn = jnp.maximum(
        jnp.dot(x, uwx_ref[...], preferred_element_type=f32)
        + jnp.dot(sm, uwm_ref[...], preferred_element_type=f32)
        + ub_ref[...], 0.0)
    xo_ref[...] = xn
    ao_ref[...] = jnp.dot(xn, wc_ref[...], preferred_element_type=f32) + b1_ref[...]
    bo_ref[...] = jnp.dot(xn, wr_ref[...], preferred_element_type=f32)


def _update_layer(x, S, inv, uwx, uwm, ub, wc, wr, b1):
    g = N // NBK
    bspec_h = pl.BlockSpec((NBK, H), lambda i: (i, 0))
    wspec = pl.BlockSpec((H, H), lambda i: (0, 0))
    return pl.pallas_call(
        _update_body,
        grid=(g,),
        in_specs=[
            bspec_h,
            pl.BlockSpec((1, NBK, H), lambda i: (0, i, 0)),
            pl.BlockSpec((1, NBK, H), lambda i: (1, i, 0)),
            pl.BlockSpec((NBK, 1), lambda i: (i, 0)),
            wspec, wspec,
            pl.BlockSpec((1, H), lambda i: (0, 0)),
            wspec, wspec,
            pl.BlockSpec((1, H), lambda i: (0, 0)),
        ],
        out_specs=(bspec_h, bspec_h, bspec_h),
        out_shape=(
            jax.ShapeDtypeStruct((N, H), f32),
            jax.ShapeDtypeStruct((N, H), f32),
            jax.ShapeDtypeStruct((N, H), f32),
        ),
    )(x, S, S, inv, uwx, uwm, ub.reshape(1, H), wc, wr, b1.reshape(1, H))


def _update_body_s0(s0_ref, s1_ref, inv_ref, x_ref,
                    uwx_ref, uwm_ref, ub_ref, xo_ref):
    sm = (s0_ref[0] + s1_ref[0]) * inv_ref[...]
    xo_ref[...] = jnp.maximum(
        jnp.dot(x_ref[...], uwx_ref[...], preferred_element_type=f32)
        + jnp.dot(sm, uwm_ref[...], preferred_element_type=f32)
        + ub_ref[...], 0.0)


def _update_last(x, S, inv, uwx, uwm, ub):
    g = N // NBK
    bspec_h = pl.BlockSpec((NBK, H), lambda i: (i, 0))
    wspec = pl.BlockSpec((H, H), lambda i: (0, 0))
    return pl.pallas_call(
        _update_body_s0,
        grid=(g,),
        in_specs=[
            pl.BlockSpec((1, NBK, H), lambda i: (0, i, 0)),
            pl.BlockSpec((1, NBK, H), lambda i: (1, i, 0)),
            pl.BlockSpec((NBK, 1), lambda i: (i, 0)),
            bspec_h,
            wspec, wspec,
            pl.BlockSpec((1, H), lambda i: (0, 0)),
        ],
        out_specs=bspec_h,
        out_shape=jax.ShapeDtypeStruct((N, H), f32),
    )(S, S, inv, x, uwx, uwm, ub.reshape(1, H))


def _readout_body(x_ref, nt_ref, batt_ref, w1_ref, b1_ref, w2_ref, b2_ref,
                  o_ref):
    ligT = (nt_ref[...] == 1).astype(f32)
    oh_bn = (lax.broadcasted_iota(jnp.int32, (NB, N), 0)
             == batt_ref[...]).astype(f32) * ligT
    lc = jnp.sum(oh_bn, axis=1, keepdims=True)
    gsum = jnp.dot(oh_bn, x_ref[...], preferred_element_type=f32)
    gmean = gsum / jnp.maximum(lc, 1.0)
    h = jnp.maximum(
        jnp.dot(gmean, w1_ref[...], preferred_element_type=f32) + b1_ref[...],
        0.0)
    o_ref[...] = jnp.dot(h, w2_ref[...], preferred_element_type=f32) + b2_ref[...]


def _readout(x, nt_t, bat_t, ro1_w, ro1_b, ro2_w, ro2_b):
    return pl.pallas_call(
        _readout_body,
        out_shape=jax.ShapeDtypeStruct((NB, 1), f32),
    )(x, nt_t, bat_t, ro1_w, ro1_b.reshape(1, H), ro2_w, ro2_b.reshape(1, 1))


def kernel(pos, z, batch, edge_index, node_type, emb, lin_in_w, lin_in_b,
           l0_m1_w, l0_m1_b, l0_m2_w, l0_m2_b, l0_u_w, l0_u_b,
           l1_m1_w, l1_m1_b, l1_m2_w, l1_m2_b, l1_u_w, l1_u_b,
           l2_m1_w, l2_m1_b, l2_m2_w, l2_m2_b, l2_u_w, l2_u_b,
           ro1_w, ro1_b, ro2_w, ro2_b):
    layers = [
        (l0_m1_w, l0_m1_b, l0_m2_w, l0_m2_b, l0_u_w, l0_u_b),
        (l1_m1_w, l1_m1_b, l1_m2_w, l1_m2_b, l1_u_w, l1_u_b),
        (l2_m1_w, l2_m1_b, l2_m2_w, l2_m2_b, l2_u_w, l2_u_b),
    ]
    row = edge_index[0].astype(jnp.int32)
    col = edge_index[1].astype(jnp.int32)
    z2 = z.astype(jnp.int32).reshape(N, 1)
    bat_n = batch.astype(jnp.int32).reshape(N, 1)
    bat_t = batch.astype(jnp.int32).reshape(1, N)
    nt_t = node_type.astype(jnp.int32).reshape(1, N)
    posT = pos.T

    we = lin_in_w[:H]
    wp = lin_in_w[H:]
    wc0, wr0, wd0 = l0_m1_w[:H], l0_m1_w[H:2 * H], l0_m1_w[2 * H]

    x, prT, A, B = _preproc(posT, z2, bat_n, bat_t, emb, we, wp,
                            lin_in_b.reshape(1, H), wc0, wr0,
                            l0_m1_b.reshape(1, H))
    d2, cntNW = _d2_counts(prT[0], prT[1], prT[2], col, row)
    dist, inv = _dist_inv(d2, cntNW.reshape(NW, N))

    import numpy as _np
    pi = _np.empty((H,), _np.int32)
    for _g in range(H // 32):
        for _k in range(16):
            pi[32 * _g + 2 * _k] = 32 * _g + _k
            pi[32 * _g + 2 * _k + 1] = 32 * _g + 16 + _k

    for li, (m1w, m1b, m2w, m2b, uw, ub) in enumerate(layers):
        wd = m1w[2 * H]
        P = _gather_layer(A, B, col, row, dist, wd)
        P_bf = lax.bitcast_convert_type(P, jnp.bfloat16).reshape(E, H)
        M = _msg_mlp2(P_bf, m2w[pi], m2b)
        S = _scatter_layer(M, row)
        uwx, uwm = uw[:H], uw[H:]
        if li < 2:
            nm1w, nm1b = layers[li + 1][0], layers[li + 1][1]
            x, A, B = _update_layer(x, S, inv, uwx, uwm, ub,
                                    nm1w[:H], nm1w[H:2 * H], nm1b)
        else:
            x = _update_last(x, S, inv, uwx, uwm, ub)

    out = _readout(x, nt_t, bat_t, ro1_w, ro1_b, ro2_w, ro2_b)
    return out.reshape(NB)


# speedup vs baseline: 20.0193x; 20.0193x over previous
import functools

import jax
import jax.numpy as jnp
from jax import lax
from jax.experimental import pallas as pl
from jax.experimental.pallas import tpu as pltpu
from jax.experimental.pallas import tpu_sc as plsc

N = 10000
E = 320000
H = 128
NB = 32
MAXZ = 100

NC = 2
NS = 16
NW = NC * NS
EPW = E // NW
K = 200
NCH = EPW // K
KS = 200
NCHS = EPW // KS
CK = 1000
NSTR = 640
NP = NS * NSTR

f32 = jnp.float32


def _preproc_body(post_ref, z_ref, batn_ref, batt_ref, emb_ref,
                  we_ref, wp_ref, lib_ref, wc_ref, wr_ref, b1_ref,
                  x0_ref, prt_ref, a0_ref, b0_ref):
    post = post_ref[...]
    z = z_ref[...]
    batn = batn_ref[...]
    batt = batt_ref[...]

    oh_nb = (lax.broadcasted_iota(jnp.int32, (N, NB), 1) == batn).astype(f32)
    oh_bn = (lax.broadcasted_iota(jnp.int32, (NB, N), 0) == batt).astype(f32)

    cnt = jnp.sum(oh_nb, axis=0, keepdims=True)
    centT = jnp.dot(post, oh_nb, preferred_element_type=f32)
    centT = centT / jnp.maximum(cnt, 1.0)
    cbT = jnp.dot(centT, oh_bn, preferred_element_type=f32)
    prT = post - cbT
    prt_ref[...] = prT

    zoh = (lax.broadcasted_iota(jnp.int32, (N, MAXZ), 1) == z).astype(f32)
    ew = jnp.dot(emb_ref[...], we_ref[...], preferred_element_type=f32)
    ez = jnp.dot(zoh, ew, preferred_element_type=f32)
    xp = lax.dot_general(prT, wp_ref[...], (((0,), (0,)), ((), ())),
                         preferred_element_type=f32)
    x0 = ez + xp + lib_ref[...]
    x0_ref[...] = x0
    a0_ref[...] = jnp.dot(x0, wc_ref[...], preferred_element_type=f32) + b1_ref[...]
    b0_ref[...] = jnp.dot(x0, wr_ref[...], preferred_element_type=f32)


def _preproc(posT, z2, bat_n, bat_t, emb, we, wp, lib, wc, wr, b1):
    return pl.pallas_call(
        _preproc_body,
        out_shape=(
            jax.ShapeDtypeStruct((N, H), f32),
            jax.ShapeDtypeStruct((3, N), f32),
            jax.ShapeDtypeStruct((N, H), f32),
            jax.ShapeDtypeStruct((N, H), f32),
        ),
    )(posT, z2, bat_n, bat_t, emb, we, wp, lib, wc, wr, b1)


def _d2_body(prx_hbm, pry_hbm, prz_hbm, col_hbm, row_hbm, zn_hbm,
             d2_hbm, cnt_hbm,
             prx, pry, prz, colv, rowv, d2v, cntv):
    cid = lax.axis_index("c")
    sid = lax.axis_index("s")
    wid = sid * NC + cid
    base = wid * EPW

    pltpu.sync_copy(prx_hbm, prx)
    pltpu.sync_copy(pry_hbm, pry)
    pltpu.sync_copy(prz_hbm, prz)
    pltpu.sync_copy(col_hbm.at[pl.ds(base, EPW)], colv)
    pltpu.sync_copy(row_hbm.at[pl.ds(base, EPW)], rowv)
    pltpu.sync_copy(zn_hbm, cntv)

    ones16 = jnp.full((16,), 1.0, f32)

    def step(g, _):
        s = pl.ds(g * 16, 16)
        ic = colv[s]
        ir = rowv[s]
        dx = plsc.load_gather(prx, [ic]) - plsc.load_gather(prx, [ir])
        dy = plsc.load_gather(pry, [ic]) - plsc.load_gather(pry, [ir])
        dz = plsc.load_gather(prz, [ic]) - plsc.load_gather(prz, [ir])
        d2v[s] = dx * dx + dy * dy + dz * dz
        plsc.addupdate_scatter(cntv, [ir], ones16)
        return 0

    lax.fori_loop(0, EPW // 16, step, 0)
    pltpu.sync_copy(d2v, d2_hbm.at[pl.ds(base, EPW)])
    pltpu.sync_copy(cntv, cnt_hbm.at[pl.ds(wid * N, N)])


def _d2_counts(prx, pry, prz, col, row):
    zn = jnp.zeros((N,), f32)
    mesh = plsc.VectorSubcoreMesh(core_axis_name="c", subcore_axis_name="s")
    return pl.kernel(
        _d2_body,
        out_type=(
            jax.ShapeDtypeStruct((E,), f32),
            jax.ShapeDtypeStruct((NW * N,), f32),
        ),
        mesh=mesh,
        scratch_types=[
            pltpu.VMEM((N,), f32),
            pltpu.VMEM((N,), f32),
            pltpu.VMEM((N,), f32),
            pltpu.VMEM((EPW,), jnp.int32),
            pltpu.VMEM((EPW,), jnp.int32),
            pltpu.VMEM((EPW,), f32),
            pltpu.VMEM((N,), f32),
        ],
        compiler_params=pltpu.CompilerParams(needs_layout_passes=False),
    )(prx, pry, prz, col, row, zn)


def _sqrt_body(d2_ref, cnt_ref, o_ref, inv_ref):
    o_ref[...] = jnp.sqrt(d2_ref[...] + 1e-12)
    c = lax.dot_general(cnt_ref[...], jnp.ones((NW, 1), f32),
                        (((0,), (0,)), ((), ())),
                        preferred_element_type=f32)
    inv_ref[...] = 1.0 / jnp.maximum(c, 1.0)


def _dist_inv(d2, cntNW):
    d2m = d2.reshape(E // 128, 128)
    out, inv = pl.pallas_call(
        _sqrt_body,
        out_shape=(
            jax.ShapeDtypeStruct((E // 128, 128), f32),
            jax.ShapeDtypeStruct((N, 1), f32),
        ),
    )(d2m, cntNW)
    return out.reshape(E), inv


HP = H // 2
bf16 = jnp.bfloat16


KG = 80
NCHG = EPW // KG


def _gather_body(a_hbm, b_hbm, col_hbm, row_hbm, dist_hbm, wd_hbm,
                 p_hbm,
                 colw, roww, distw, av0, bv0, av1, bv1,
                 wdv, sg0, sg1, so0, so1):
    cid = lax.axis_index("c")
    sid = lax.axis_index("s")
    wid = sid * NC + cid

    av = (av0, av1)
    bv = (bv0, bv1)
    sg = (sg0, sg1)
    so = (so0, so1)

    pltpu.sync_copy(wd_hbm, wdv)
    pltpu.sync_copy(col_hbm.at[pl.ds(wid * EPW, EPW)], colw)
    pltpu.sync_copy(row_hbm.at[pl.ds(wid * EPW, EPW)], roww)
    pltpu.sync_copy(dist_hbm.at[pl.ds(wid * EPW, EPW)], distw)

    def fetch(ch, b):
        base = ch * KG
        pltpu.async_copy(a_hbm.at[colw.at[pl.ds(base, KG)]], av[b], sg[b])
        pltpu.async_copy(b_hbm.at[roww.at[pl.ds(base, KG)]], bv[b], sg[b])

    def edge(b, e, ro, co, d):
        del ro, co
        for gg in range(H // 16):
            s = pl.ds(gg * 16, 16)
            bv[b][e, s] = jnp.maximum(
                av[b][e, s] + bv[b][e, s] + d * wdv[s], 0.0)

    def compute(b, ch):
        def group(g, _):
            dvec = distw[pl.ds(ch * KG + g * 16, 16)]
            for j in range(16):
                edge(b, g * 16 + j, g * 8 + j // 2, (j % 2) * 64, dvec[j])
            return 0

        lax.fori_loop(0, KG // 16, group, 0)

    def wait_gather(b):
        pltpu.make_async_copy(
            a_hbm.at[colw.at[pl.ds(0, KG)]], av[b], sg[b]).wait()
        pltpu.make_async_copy(
            b_hbm.at[roww.at[pl.ds(0, KG)]], bv[b], sg[b]).wait()

    def wait_out(b):
        pltpu.make_async_copy(bv[b], p_hbm.at[pl.ds(0, KG)], so[b]).wait()

    def emit(b, ch):
        base = pl.multiple_of(wid * EPW + ch * KG, 8)
        pltpu.async_copy(bv[b], p_hbm.at[pl.ds(base, KG)], so[b])

    fetch(0, 0)

    def step2(g2, _):
        for b in range(2):
            ch = g2 * 2 + b
            nb = 1 - b

            @pl.when(ch + 1 < NCHG)
            def _():
                @pl.when(ch >= 1)
                def _():
                    wait_out(nb)
                fetch(ch + 1, nb)

            wait_gather(b)
            compute(b, ch)
            emit(b, ch)
        return 0

    lax.fori_loop(0, NCHG // 2, step2, 0)
    if NCHG % 2:
        wait_gather(0)
        compute(0, NCHG - 1)
        emit(0, NCHG - 1)
    wait_out(0)
    wait_out(1)


def _gather_layer(A, B, col, row, dist, wd):
    mesh = plsc.VectorSubcoreMesh(core_axis_name="c", subcore_axis_name="s")
    return pl.kernel(
        _gather_body,
        out_type=jax.ShapeDtypeStruct((E, H), f32),
        mesh=mesh,
        scratch_types=[
            pltpu.VMEM((EPW,), jnp.int32),
            pltpu.VMEM((EPW,), jnp.int32),
            pltpu.VMEM((EPW,), f32),
            pltpu.VMEM((KG, H), f32),
            pltpu.VMEM((KG, H), f32),
            pltpu.VMEM((KG, H), f32),
            pltpu.VMEM((KG, H), f32),
            pltpu.VMEM((H,), f32),
            pltpu.SemaphoreType.DMA,
            pltpu.SemaphoreType.DMA,
            pltpu.SemaphoreType.DMA,
            pltpu.SemaphoreType.DMA,
        ],
        compiler_params=pltpu.CompilerParams(needs_layout_passes=False),
    )(A, B, col, row, dist, wd)


BE = 3200


def _mm_body(p_ref, w_ref, b_ref, o_ref):
    o_ref[...] = jnp.maximum(
        jnp.dot(p_ref[...].astype(f32), w_ref[...],
                preferred_element_type=f32) + b_ref[...],
        0.0)


def _msg_mlp2(P, w, b):
    return pl.pallas_call(
        _mm_body,
        grid=(E // BE,),
        in_specs=[
            pl.BlockSpec((BE, H), lambda i: (i, 0)),
            pl.BlockSpec((H, H), lambda i: (0, 0)),
            pl.BlockSpec((1, H), lambda i: (0, 0)),
        ],
        out_specs=pl.BlockSpec((BE, H), lambda i: (i, 0)),
        out_shape=jax.ShapeDtypeStruct((E, H), f32),
    )(P, w, b.reshape(1, H))


def _scatter_body(m_hbm, row_hbm, zrows_hbm,
                  s_hbm,
                  rowv, mv, shared):
    cid = lax.axis_index("c")
    sid = lax.axis_index("s")
    wid = sid * NC + cid

    pltpu.sync_copy(zrows_hbm, shared.at[pl.ds(sid * NSTR, NSTR)])
    plsc.subcore_barrier()

    def chunk(ch, _):
        base = wid * EPW + ch * KS
        pltpu.sync_copy(row_hbm.at[pl.ds(base, KS)], rowv)
        pltpu.sync_copy(m_hbm.at[pl.ds(base, KS)], mv)
        pltpu.sync_copy(mv, shared.at[rowv], add=True)
        return 0

    lax.fori_loop(0, NCHS, chunk, 0)
    plsc.subcore_barrier()
    pltpu.sync_copy(shared.at[pl.ds(sid * NSTR, NSTR)],
                    s_hbm.at[cid, pl.ds(sid * NSTR, NSTR)])


def _scatter_layer(M, row):
    zrows = jnp.zeros((NSTR, H), f32)
    mesh = plsc.VectorSubcoreMesh(core_axis_name="c", subcore_axis_name="s")
    return pl.kernel(
        _scatter_body,
        out_type=jax.ShapeDtypeStruct((NC, NP, H), f32),
        mesh=mesh,
        scratch_types=[
            pltpu.VMEM((KS,), jnp.int32),
            pltpu.VMEM((KS, H), f32),
            pltpu.VMEM_SHARED((NP, H), f32),
        ],
    )(M, row, zrows)


NBK = 2000


def _update_body(x_ref, s0_ref, s1_ref, inv_ref,
                 uwx_ref, uwm_ref, ub_ref, wc_ref, wr_ref, b1_ref,
                 xo_ref, ao_ref, bo_ref):
    sm = (s0_ref[0] + s1_ref[0]) * inv_ref[...]
    x = x_ref[...]
    xn = jnp.maximum(
        jnp.dot(x, uwx_ref[...], preferred_element_type=f32)
        + jnp.dot(sm, uwm_ref[...], preferred_element_type=f32)
        + ub_ref[...], 0.0)
    xo_ref[...] = xn
    ao_ref[...] = jnp.dot(xn, wc_ref[...], preferred_element_type=f32) + b1_ref[...]
    bo_ref[...] = jnp.dot(xn, wr_ref[...], preferred_element_type=f32)


def _update_layer(x, S, inv, uwx, uwm, ub, wc, wr, b1):
    g = N // NBK
    bspec_h = pl.BlockSpec((NBK, H), lambda i: (i, 0))
    wspec = pl.BlockSpec((H, H), lambda i: (0, 0))
    return pl.pallas_call(
        _update_body,
        grid=(g,),
        in_specs=[
            bspec_h,
            pl.BlockSpec((1, NBK, H), lambda i: (0, i, 0)),
            pl.BlockSpec((1, NBK, H), lambda i: (1, i, 0)),
            pl.BlockSpec((NBK, 1), lambda i: (i, 0)),
            wspec, wspec,
            pl.BlockSpec((1, H), lambda i: (0, 0)),
            wspec, wspec,
            pl.BlockSpec((1, H), lambda i: (0, 0)),
        ],
        out_specs=(bspec_h, bspec_h, bspec_h),
        out_shape=(
            jax.ShapeDtypeStruct((N, H), f32),
            jax.ShapeDtypeStruct((N, H), f32),
            jax.ShapeDtypeStruct((N, H), f32),
        ),
    )(x, S, S, inv, uwx, uwm, ub.reshape(1, H), wc, wr, b1.reshape(1, H))


def _update_body_s0(s0_ref, s1_ref, inv_ref, x_ref,
                    uwx_ref, uwm_ref, ub_ref, xo_ref):
    sm = (s0_ref[0] + s1_ref[0]) * inv_ref[...]
    xo_ref[...] = jnp.maximum(
        jnp.dot(x_ref[...], uwx_ref[...], preferred_element_type=f32)
        + jnp.dot(sm, uwm_ref[...], preferred_element_type=f32)
        + ub_ref[...], 0.0)


def _update_last(x, S, inv, uwx, uwm, ub):
    g = N // NBK
    bspec_h = pl.BlockSpec((NBK, H), lambda i: (i, 0))
    wspec = pl.BlockSpec((H, H), lambda i: (0, 0))
    return pl.pallas_call(
        _update_body_s0,
        grid=(g,),
        in_specs=[
            pl.BlockSpec((1, NBK, H), lambda i: (0, i, 0)),
            pl.BlockSpec((1, NBK, H), lambda i: (1, i, 0)),
            pl.BlockSpec((NBK, 1), lambda i: (i, 0)),
            bspec_h,
            wspec, wspec,
            pl.BlockSpec((1, H), lambda i: (0, 0)),
        ],
        out_specs=bspec_h,
        out_shape=jax.ShapeDtypeStruct((N, H), f32),
    )(S, S, inv, x, uwx, uwm, ub.reshape(1, H))


def _readout_body(x_ref, nt_ref, batt_ref, w1_ref, b1_ref, w2_ref, b2_ref,
                  o_ref):
    ligT = (nt_ref[...] == 1).astype(f32)
    oh_bn = (lax.broadcasted_iota(jnp.int32, (NB, N), 0)
             == batt_ref[...]).astype(f32) * ligT
    lc = jnp.sum(oh_bn, axis=1, keepdims=True)
    gsum = jnp.dot(oh_bn, x_ref[...], preferred_element_type=f32)
    gmean = gsum / jnp.maximum(lc, 1.0)
    h = jnp.maximum(
        jnp.dot(gmean, w1_ref[...], preferred_element_type=f32) + b1_ref[...],
        0.0)
    o_ref[...] = jnp.dot(h, w2_ref[...], preferred_element_type=f32) + b2_ref[...]


def _readout(x, nt_t, bat_t, ro1_w, ro1_b, ro2_w, ro2_b):
    return pl.pallas_call(
        _readout_body,
        out_shape=jax.ShapeDtypeStruct((NB, 1), f32),
    )(x, nt_t, bat_t, ro1_w, ro1_b.reshape(1, H), ro2_w, ro2_b.reshape(1, 1))


def kernel(pos, z, batch, edge_index, node_type, emb, lin_in_w, lin_in_b,
           l0_m1_w, l0_m1_b, l0_m2_w, l0_m2_b, l0_u_w, l0_u_b,
           l1_m1_w, l1_m1_b, l1_m2_w, l1_m2_b, l1_u_w, l1_u_b,
           l2_m1_w, l2_m1_b, l2_m2_w, l2_m2_b, l2_u_w, l2_u_b,
           ro1_w, ro1_b, ro2_w, ro2_b):
    layers = [
        (l0_m1_w, l0_m1_b, l0_m2_w, l0_m2_b, l0_u_w, l0_u_b),
        (l1_m1_w, l1_m1_b, l1_m2_w, l1_m2_b, l1_u_w, l1_u_b),
        (l2_m1_w, l2_m1_b, l2_m2_w, l2_m2_b, l2_u_w, l2_u_b),
    ]
    row = edge_index[0].astype(jnp.int32)
    col = edge_index[1].astype(jnp.int32)
    z2 = z.astype(jnp.int32).reshape(N, 1)
    bat_n = batch.astype(jnp.int32).reshape(N, 1)
    bat_t = batch.astype(jnp.int32).reshape(1, N)
    nt_t = node_type.astype(jnp.int32).reshape(1, N)
    posT = pos.T

    we = lin_in_w[:H]
    wp = lin_in_w[H:]
    wc0, wr0, wd0 = l0_m1_w[:H], l0_m1_w[H:2 * H], l0_m1_w[2 * H]

    x, prT, A, B = _preproc(posT, z2, bat_n, bat_t, emb, we, wp,
                            lin_in_b.reshape(1, H), wc0, wr0,
                            l0_m1_b.reshape(1, H))
    d2, cntNW = _d2_counts(prT[0], prT[1], prT[2], col, row)
    dist, inv = _dist_inv(d2, cntNW.reshape(NW, N))

    for li, (m1w, m1b, m2w, m2b, uw, ub) in enumerate(layers):
        wd = m1w[2 * H]
        P = _gather_layer(A, B, col, row, dist, wd)
        M = _msg_mlp2(P, m2w, m2b)
        S = _scatter_layer(M, row)
        uwx, uwm = uw[:H], uw[H:]
        if li < 2:
            nm1w, nm1b = layers[li + 1][0], layers[li + 1][1]
            x, A, B = _update_layer(x, S, inv, uwx, uwm, ub,
                                    nm1w[:H], nm1w[H:2 * H], nm1b)
        else:
            x = _update_last(x, S, inv, uwx, uwm, ub)

    out = _readout(x, nt_t, bat_t, ro1_w, ro1_b, ro2_w, ro2_b)
    return out.reshape(NB)
